# Initial kernel scaffold; baseline (speedup 1.0000x reference)
import functools

import jax
import jax.numpy as jnp
from jax import lax
from jax.experimental import pallas as pl
from jax.experimental.pallas import tpu as pltpu
from jax.experimental.pallas import tpu_sc as plsc

NG = 10000
ND = 2000
DIN = 512
DH = 128
L = 2
WD = 136

NC = 2
NS = 16
CHUNK = 80


@functools.lru_cache(maxsize=None)
def _sc_segment_sum(E, NSRC, NDST):
    EW = E // (NC * NS)
    assert EW % CHUNK == 0 and EW % 8 == 0
    n_chunks = EW // CHUNK
    rows_per_tile = NDST // NS
    mesh = plsc.VectorSubcoreMesh(
        core_axis_name="c", subcore_axis_name="s", num_cores=NC, num_subcores=NS
    )

    @functools.partial(
        pl.kernel,
        out_type=jax.ShapeDtypeStruct((NC, NDST, WD), jnp.float32),
        mesh=mesh,
        scratch_types=[
            pltpu.VMEM((CHUNK,), jnp.int32),
            pltpu.VMEM((CHUNK,), jnp.int32),
            pltpu.VMEM((CHUNK, WD), jnp.float32),
            pltpu.VMEM_SHARED((NDST, WD), jnp.float32),
            pltpu.SemaphoreType.DMA,
        ],
    )
    def seg_sum(x_hbm, edge_hbm, zeros_hbm, out_hbm, sidx, didx, rows, acc, sem):
        c = lax.axis_index("c")
        s = lax.axis_index("s")
        r0 = s * rows_per_tile
        pltpu.sync_copy(
            zeros_hbm.at[pl.ds(r0, rows_per_tile)],
            acc.at[pl.ds(r0, rows_per_tile)],
        )
        plsc.subcore_barrier()
        base = (c * NS + s) * EW

        def body(j, carry):
            off = base + j * CHUNK
            pltpu.sync_copy(edge_hbm.at[0, pl.ds(off, CHUNK)], sidx)
            pltpu.sync_copy(edge_hbm.at[1, pl.ds(off, CHUNK)], didx)
            pltpu.async_copy(x_hbm.at[sidx], rows, sem).wait()
            pltpu.sync_copy(rows, acc.at[didx], add=True)
            return carry

        lax.fori_loop(0, n_chunks, body, 0)
        plsc.subcore_barrier()
        pltpu.sync_copy(
            acc.at[pl.ds(r0, rows_per_tile)],
            out_hbm.at[c, pl.ds(r0, rows_per_tile)],
        )

    return seg_sum


def _proj_body(x_ref, w_ref, b_ref, o_ref):
    y = jnp.dot(x_ref[...], w_ref[...], preferred_element_type=jnp.float32)
    y = y + b_ref[...]
    r = y.shape[0]
    pad = jnp.concatenate(
        [jnp.ones((r, 1), jnp.float32), jnp.zeros((r, WD - DH - 1), jnp.float32)],
        axis=1,
    )
    o_ref[...] = jnp.concatenate([y, pad], axis=1)


def _project(x, w, b, block_rows):
    n = x.shape[0]
    grid = n // block_rows
    return pl.pallas_call(
        _proj_body,
        grid=(grid,),
        in_specs=[
            pl.BlockSpec((block_rows, DIN), lambda i: (i, 0)),
            pl.BlockSpec((DIN, DH), lambda i: (0, 0)),
            pl.BlockSpec((1, DH), lambda i: (0, 0)),
        ],
        out_specs=pl.BlockSpec((block_rows, WD), lambda i: (i, 0)),
        out_shape=jax.ShapeDtypeStruct((n, WD), jnp.float32),
    )(x, w, b)


def _layer_body(T):
    def body(*refs):
        p_refs = refs[:T]
        x_ref, wn_ref, wr_ref, b_ref, o_ref = refs[T:]
        xroot = x_ref[:, :DH]
        wr_sum = jnp.sum(wr_ref[...], axis=0)
        acc = jnp.dot(xroot, wr_sum, preferred_element_type=jnp.float32)
        acc = acc + jnp.sum(b_ref[...], axis=0)[None, :]
        for t in range(T):
            p = p_refs[t][...]
            ssum = p[0] + p[1]
            cnt = jnp.maximum(ssum[:, DH : DH + 1], 1.0)
            agg = ssum[:, :DH] * (1.0 / cnt)
            acc = acc + jnp.dot(agg, wn_ref[t], preferred_element_type=jnp.float32)
        y = jnp.maximum(acc, 0.0)
        r = y.shape[0]
        pad = jnp.concatenate(
            [jnp.ones((r, 1), jnp.float32), jnp.zeros((r, WD - DH - 1), jnp.float32)],
            axis=1,
        )
        o_ref[...] = jnp.concatenate([y, pad], axis=1)

    return body


def _layer(partials, x, wn, wr, b, block_rows):
    T = len(partials)
    n = x.shape[0]
    grid = n // block_rows
    p_specs = [
        pl.BlockSpec((NC, block_rows, WD), lambda i: (0, i, 0)) for _ in range(T)
    ]
    return pl.pallas_call(
        _layer_body(T),
        grid=(grid,),
        in_specs=p_specs
        + [
            pl.BlockSpec((block_rows, WD), lambda i: (i, 0)),
            pl.BlockSpec((T, DH, DH), lambda i: (0, 0, 0)),
            pl.BlockSpec((T, DH, DH), lambda i: (0, 0, 0)),
            pl.BlockSpec((T, DH), lambda i: (0, 0)),
        ],
        out_specs=pl.BlockSpec((block_rows, WD), lambda i: (i, 0)),
        out_shape=jax.ShapeDtypeStruct((n, WD), jnp.float32),
    )(*partials, x, wn, wr, b)


def kernel(x_gene, x_drug, edge_index_ppi, edge_index_gsea, edge_index_pcc,
           edge_index_dds, edge_index_dti_dg, edge_index_dti_gd,
           Wg, bg, Wd, bd, Wl, bl, Wr):
    zeros_g = jnp.zeros((NG, WD), jnp.float32)
    zeros_d = jnp.zeros((ND, WD), jnp.float32)

    xg = _project(x_gene, Wg, bg.reshape(1, DH), 2000)
    xd = _project(x_drug, Wd, bd.reshape(1, DH), 2000)

    seg_gg = _sc_segment_sum(320000, NG, NG)
    seg_dg = _sc_segment_sum(64000, ND, NG)
    seg_dd = _sc_segment_sum(64000, ND, ND)
    seg_gd = _sc_segment_sum(64000, NG, ND)

    for l in range(L):
        p_ppi = seg_gg(xg, edge_index_ppi, zeros_g)
        p_gsea = seg_gg(xg, edge_index_gsea, zeros_g)
        p_pcc = seg_gg(xg, edge_index_pcc, zeros_g)
        p_dti_dg = seg_dg(xd, edge_index_dti_dg, zeros_g)
        p_dds = seg_dd(xd, edge_index_dds, zeros_d)
        p_dti_gd = seg_gd(xg, edge_index_dti_gd, zeros_d)

        wn_g = jnp.stack([Wl[l, 0], Wl[l, 1], Wl[l, 2], Wl[l, 4]])
        wr_g = jnp.stack([Wr[l, 0], Wr[l, 1], Wr[l, 2], Wr[l, 4]])
        b_g = jnp.stack([bl[l, 0], bl[l, 1], bl[l, 2], bl[l, 4]])
        wn_d = jnp.stack([Wl[l, 3], Wl[l, 5]])
        wr_d = jnp.stack([Wr[l, 3], Wr[l, 5]])
        b_d = jnp.stack([bl[l, 3], bl[l, 5]])

        xg = _layer([p_ppi, p_gsea, p_pcc, p_dti_dg], xg, wn_g, wr_g, b_g, 2000)
        xd = _layer([p_dds, p_dti_gd], xd, wn_d, wr_d, b_d, 2000)

    return xg[:, :DH], xd[:, :DH]


# speedup vs baseline: 3.0883x; 3.0883x over previous
import functools

import jax
import jax.numpy as jnp
from jax import lax
from jax.experimental import pallas as pl
from jax.experimental.pallas import tpu as pltpu
from jax.experimental.pallas import tpu_sc as plsc

NG = 10000
ND = 2000
DIN = 512
DH = 128
L = 2

NC = 2
NS = 16
CHUNK = 80
RB = 200

_MESH = dict(core_axis_name="c", subcore_axis_name="s", num_cores=NC,
             num_subcores=NS)


def _strided_row_copy(s, n_rblk, do_copy):
    n_rpass = -(-n_rblk // NS)
    for i in range(n_rpass):
        blk = s + NS * i

        @pl.when(blk < n_rblk)
        def _():
            do_copy(blk * RB)


@functools.lru_cache(maxsize=None)
def _sc_segment_sum(E, NSRC, NDST):
    EW = E // (NC * NS)
    assert EW % CHUNK == 0
    n_chunks = EW // CHUNK
    assert NDST % RB == 0
    n_rblk = NDST // RB

    @functools.partial(
        pl.kernel,
        out_type=jax.ShapeDtypeStruct((NC, NDST, DH), jnp.float32),
        mesh=plsc.VectorSubcoreMesh(**_MESH),
        scratch_types=[
            pltpu.VMEM((CHUNK,), jnp.int32),
            pltpu.VMEM((CHUNK,), jnp.int32),
            pltpu.VMEM((CHUNK, DH), jnp.float32),
            pltpu.VMEM_SHARED((NDST, DH), jnp.float32),
            pltpu.SemaphoreType.DMA,
        ],
    )
    def seg_sum(x_hbm, edge_hbm, zeros_hbm, out_hbm, sidx, didx, rows, acc, sem):
        c = lax.axis_index("c")
        s = lax.axis_index("s")
        _strided_row_copy(s, n_rblk, lambda r0: pltpu.sync_copy(
            zeros_hbm.at[pl.ds(r0, RB)], acc.at[pl.ds(r0, RB)]))
        plsc.subcore_barrier()
        base = (c * NS + s) * EW

        def body(j, carry):
            off = base + j * CHUNK
            pltpu.sync_copy(edge_hbm.at[pl.ds(off, CHUNK)], sidx)
            pltpu.sync_copy(edge_hbm.at[pl.ds(E + off, CHUNK)], didx)
            pltpu.async_copy(x_hbm.at[sidx], rows, sem).wait()
            pltpu.sync_copy(rows, acc.at[didx], add=True)
            return carry

        lax.fori_loop(0, n_chunks, body, 0)
        plsc.subcore_barrier()
        _strided_row_copy(s, n_rblk, lambda r0: pltpu.sync_copy(
            acc.at[pl.ds(r0, RB)], out_hbm.at[c, pl.ds(r0, RB)]))

    return seg_sum


@functools.lru_cache(maxsize=None)
def _sc_count(E, NDST):
    EW = E // (NC * NS)
    assert EW % CHUNK == 0
    n_chunks = EW // CHUNK
    assert NDST % RB == 0
    n_rblk = NDST // RB

    @functools.partial(
        pl.kernel,
        out_type=jax.ShapeDtypeStruct((NC, NDST, DH), jnp.float32),
        mesh=plsc.VectorSubcoreMesh(**_MESH),
        scratch_types=[
            pltpu.VMEM((CHUNK,), jnp.int32),
            pltpu.VMEM((CHUNK, DH), jnp.float32),
            pltpu.VMEM_SHARED((NDST, DH), jnp.float32),
        ],
    )
    def count(edge_hbm, zeros_hbm, ones_hbm, out_hbm, didx, ones_v, acc):
        c = lax.axis_index("c")
        s = lax.axis_index("s")
        pltpu.sync_copy(ones_hbm, ones_v)
        _strided_row_copy(s, n_rblk, lambda r0: pltpu.sync_copy(
            zeros_hbm.at[pl.ds(r0, RB)], acc.at[pl.ds(r0, RB)]))
        plsc.subcore_barrier()
        base = (c * NS + s) * EW

        def body(j, carry):
            off = base + j * CHUNK
            pltpu.sync_copy(edge_hbm.at[pl.ds(E + off, CHUNK)], didx)
            pltpu.sync_copy(ones_v, acc.at[didx], add=True)
            return carry

        lax.fori_loop(0, n_chunks, body, 0)
        plsc.subcore_barrier()
        _strided_row_copy(s, n_rblk, lambda r0: pltpu.sync_copy(
            acc.at[pl.ds(r0, RB)], out_hbm.at[c, pl.ds(r0, RB)]))

    return count


def _proj_body(x_ref, w_ref, b_ref, o_ref):
    y = jnp.dot(x_ref[...], w_ref[...], preferred_element_type=jnp.float32)
    o_ref[...] = y + b_ref[...]


def _project(x, w, b, block_rows):
    n = x.shape[0]
    return pl.pallas_call(
        _proj_body,
        grid=(n // block_rows,),
        in_specs=[
            pl.BlockSpec((block_rows, DIN), lambda i: (i, 0)),
            pl.BlockSpec((DIN, DH), lambda i: (0, 0)),
            pl.BlockSpec((1, DH), lambda i: (0, 0)),
        ],
        out_specs=pl.BlockSpec((block_rows, DH), lambda i: (i, 0)),
        out_shape=jax.ShapeDtypeStruct((n, DH), jnp.float32),
    )(x, w, b)


def _layer_body(T):
    def body(*refs):
        p_refs = refs[:T]
        c_refs = refs[T:2 * T]
        x_ref, wn_ref, wr_ref, b_ref, o_ref = refs[2 * T:]
        wr_sum = jnp.sum(wr_ref[...], axis=0)
        acc = jnp.dot(x_ref[...], wr_sum, preferred_element_type=jnp.float32)
        acc = acc + jnp.sum(b_ref[...], axis=0)[None, :]
        for t in range(T):
            p = p_refs[t][...]
            ssum = p[0] + p[1]
            cnt = c_refs[t][0, :, 0:1] + c_refs[t][1, :, 0:1]
            agg = ssum * (1.0 / jnp.maximum(cnt, 1.0))
            acc = acc + jnp.dot(agg, wn_ref[t], preferred_element_type=jnp.float32)
        o_ref[...] = jnp.maximum(acc, 0.0)

    return body


def _layer(partials, counts, x, wn, wr, b, block_rows):
    T = len(partials)
    n = x.shape[0]
    pc_specs = [
        pl.BlockSpec((NC, block_rows, DH), lambda i: (0, i, 0))
        for _ in range(2 * T)
    ]
    return pl.pallas_call(
        _layer_body(T),
        grid=(n // block_rows,),
        in_specs=pc_specs
        + [
            pl.BlockSpec((block_rows, DH), lambda i: (i, 0)),
            pl.BlockSpec((T, DH, DH), lambda i: (0, 0, 0)),
            pl.BlockSpec((T, DH, DH), lambda i: (0, 0, 0)),
            pl.BlockSpec((T, DH), lambda i: (0, 0)),
        ],
        out_specs=pl.BlockSpec((block_rows, DH), lambda i: (i, 0)),
        out_shape=jax.ShapeDtypeStruct((n, DH), jnp.float32),
    )(*partials, *counts, x, wn, wr, b)


def kernel(x_gene, x_drug, edge_index_ppi, edge_index_gsea, edge_index_pcc,
           edge_index_dds, edge_index_dti_dg, edge_index_dti_gd,
           Wg, bg, Wd, bd, Wl, bl, Wr):
    zeros_g = jnp.zeros((NG, DH), jnp.float32)
    zeros_d = jnp.zeros((ND, DH), jnp.float32)
    ones_c = jnp.ones((CHUNK, DH), jnp.float32)

    xg = _project(x_gene, Wg, bg.reshape(1, DH), 2000)
    xd = _project(x_drug, Wd, bd.reshape(1, DH), 2000)

    seg_gg = _sc_segment_sum(320000, NG, NG)
    seg_dg = _sc_segment_sum(64000, ND, NG)
    seg_dd = _sc_segment_sum(64000, ND, ND)
    seg_gd = _sc_segment_sum(64000, NG, ND)
    cnt_big = _sc_count(320000, NG)
    cnt_g = _sc_count(64000, NG)
    cnt_d = _sc_count(64000, ND)

    e_ppi = edge_index_ppi.reshape(-1)
    e_gsea = edge_index_gsea.reshape(-1)
    e_pcc = edge_index_pcc.reshape(-1)
    e_dds = edge_index_dds.reshape(-1)
    e_dti_dg = edge_index_dti_dg.reshape(-1)
    e_dti_gd = edge_index_dti_gd.reshape(-1)

    c_ppi = cnt_big(e_ppi, zeros_g, ones_c)
    c_gsea = cnt_big(e_gsea, zeros_g, ones_c)
    c_pcc = cnt_big(e_pcc, zeros_g, ones_c)
    c_dti_dg = cnt_g(e_dti_dg, zeros_g, ones_c)
    c_dds = cnt_d(e_dds, zeros_d, ones_c)
    c_dti_gd = cnt_d(e_dti_gd, zeros_d, ones_c)

    for l in range(L):
        p_ppi = seg_gg(xg, e_ppi, zeros_g)
        p_gsea = seg_gg(xg, e_gsea, zeros_g)
        p_pcc = seg_gg(xg, e_pcc, zeros_g)
        p_dti_dg = seg_dg(xd, e_dti_dg, zeros_g)
        p_dds = seg_dd(xd, e_dds, zeros_d)
        p_dti_gd = seg_gd(xg, e_dti_gd, zeros_d)

        wn_g = jnp.stack([Wl[l, 0], Wl[l, 1], Wl[l, 2], Wl[l, 4]])
        wr_g = jnp.stack([Wr[l, 0], Wr[l, 1], Wr[l, 2], Wr[l, 4]])
        b_g = jnp.stack([bl[l, 0], bl[l, 1], bl[l, 2], bl[l, 4]])
        wn_d = jnp.stack([Wl[l, 3], Wl[l, 5]])
        wr_d = jnp.stack([Wr[l, 3], Wr[l, 5]])
        b_d = jnp.stack([bl[l, 3], bl[l, 5]])

        xg = _layer([p_ppi, p_gsea, p_pcc, p_dti_dg],
                    [c_ppi, c_gsea, c_pcc, c_dti_dg],
                    xg, wn_g, wr_g, b_g, 2000)
        xd = _layer([p_dds, p_dti_gd], [c_dds, c_dti_gd],
                    xd, wn_d, wr_d, b_d, 2000)

    return xg, xd


# speedup vs baseline: 6.6218x; 2.1442x over previous
import functools

import jax
import jax.numpy as jnp
from jax import lax
from jax.experimental import pallas as pl
from jax.experimental.pallas import tpu as pltpu
from jax.experimental.pallas import tpu_sc as plsc

NG = 10000
ND = 2000
DIN = 512
DH = 128
L = 2

NC = 2
NS = 16
CHUNK = 80
RB = 200

_MESH = dict(core_axis_name="c", subcore_axis_name="s", num_cores=NC,
             num_subcores=NS)


def _strided_row_copy(s, n_rblk, do_copy):
    n_rpass = -(-n_rblk // NS)
    for i in range(n_rpass):
        blk = s + NS * i

        @pl.when(blk < n_rblk)
        def _():
            do_copy(blk * RB)


@functools.lru_cache(maxsize=None)
def _sc_segment_sum(E, NSRC, NDST):
    NW = NC * NS
    EW = E // NW
    assert EW % CHUNK == 0
    n_chunks = EW // CHUNK
    assert n_chunks % 4 == 1
    n_bodies = (n_chunks - 1) // 4
    assert NDST % RB == 0
    n_rblk = NDST // RB

    @functools.partial(
        pl.kernel,
        out_type=jax.ShapeDtypeStruct((NC, NDST, DH), jnp.float32),
        mesh=plsc.VectorSubcoreMesh(**_MESH),
        scratch_types=[
            pltpu.VMEM((4 * CHUNK,), jnp.int32),
            pltpu.VMEM((n_chunks, CHUNK), jnp.int32),
            pltpu.VMEM((2, CHUNK, DH), jnp.float32),
            pltpu.VMEM_SHARED((NDST, DH), jnp.float32),
            pltpu.SemaphoreType.DMA,
            pltpu.SemaphoreType.DMA,
            pltpu.SemaphoreType.DMA,
            pltpu.SemaphoreType.DMA,
            pltpu.SemaphoreType.DMA,
            pltpu.SemaphoreType.DMA,
            pltpu.SemaphoreType.DMA,
            pltpu.SemaphoreType.DMA,
        ],
    )
    def seg_sum(x_hbm, src_hbm, dst_hbm, zeros_hbm, out_hbm,
                sidx, didx, rows, acc, gsA, gsB, ssA, ssB, is0, is1, is2, is3):
        c = lax.axis_index("c")
        s = lax.axis_index("s")
        wid = c * NS + s
        isem = [is0, is1, is2, is3]
        pltpu.sync_copy(dst_hbm.at[wid], didx)
        _strided_row_copy(s, n_rblk, lambda r0: pltpu.sync_copy(
            zeros_hbm.at[pl.ds(r0, RB)], acc.at[pl.ds(r0, RB)]))
        plsc.subcore_barrier()
        ebase = wid * EW

        def slot(sl):
            return sidx.at[pl.ds(sl * CHUNK, CHUNK)]

        def fire_idx(x, sl, guard=True):
            def go():
                pltpu.async_copy(src_hbm.at[pl.ds(ebase + x * CHUNK, CHUNK)],
                                 slot(sl), isem[sl])
            if guard is True:
                go()
            else:
                pl.when(guard)(go)

        def w_idx(sl, guard=True):
            def go():
                pltpu.make_async_copy(
                    src_hbm.at[pl.ds(ebase, CHUNK)], slot(sl),
                    isem[sl]).wait()
            if guard is True:
                go()
            else:
                pl.when(guard)(go)

        def fire_gather(sl, buf, sem, guard=True):
            def go():
                pltpu.async_copy(x_hbm.at[slot(sl)], rows.at[buf], sem)
            if guard is True:
                go()
            else:
                pl.when(guard)(go)

        def w_gather(buf, sem):
            pltpu.make_async_copy(x_hbm.at[slot(0)], rows.at[buf], sem).wait()

        def fire_scatter(x, buf, sem):
            pltpu.async_copy(rows.at[buf], acc.at[didx.at[x]], sem, add=True)

        def w_scatter(buf, sem):
            pltpu.make_async_copy(rows.at[buf], acc.at[didx.at[0]],
                                  sem).wait()

        A, B = 0, 1
        for k in range(4):
            fire_idx(k, k)
        w_idx(0)
        fire_gather(0, B, gsB)
        w_gather(B, gsB)
        fire_idx(4, 0)
        fire_scatter(0, B, ssB)
        w_idx(1)
        fire_gather(1, A, gsA)

        def body(h, carry):
            q = 1 + 4 * h
            w_idx(2)
            w_scatter(B, ssB)
            fire_gather(2, B, gsB)
            w_gather(A, gsA)
            fire_idx(q + 4, 1, q + 4 < n_chunks)
            fire_scatter(q, A, ssA)

            w_idx(3)
            w_scatter(A, ssA)
            fire_gather(3, A, gsA)
            w_gather(B, gsB)
            fire_idx(q + 5, 2, q + 5 < n_chunks)
            fire_scatter(q + 1, B, ssB)

            w_idx(0)
            w_scatter(B, ssB)
            fire_gather(0, B, gsB)
            w_gather(A, gsA)
            fire_idx(q + 6, 3, q + 6 < n_chunks)
            fire_scatter(q + 2, A, ssA)

            w_idx(1, q + 4 < n_chunks)
            w_scatter(A, ssA)
            fire_gather(1, A, gsA, q + 4 < n_chunks)
            w_gather(B, gsB)
            fire_idx(q + 7, 0, q + 7 < n_chunks)
            fire_scatter(q + 3, B, ssB)
            return carry

        lax.fori_loop(0, n_bodies, body, 0)
        w_scatter(B, ssB)
        plsc.subcore_barrier()
        _strided_row_copy(s, n_rblk, lambda r0: pltpu.sync_copy(
            acc.at[pl.ds(r0, RB)], out_hbm.at[c, pl.ds(r0, RB)]))

    return seg_sum


@functools.lru_cache(maxsize=None)
def _sc_count(E, NDST):
    NW = NC * NS
    EW = E // NW
    assert EW % CHUNK == 0
    n_chunks = EW // CHUNK
    assert n_chunks % 2 == 1
    n_pairs = (n_chunks - 1) // 2
    assert NDST % RB == 0
    n_rblk = NDST // RB

    @functools.partial(
        pl.kernel,
        out_type=jax.ShapeDtypeStruct((NC, NDST, DH), jnp.float32),
        mesh=plsc.VectorSubcoreMesh(**_MESH),
        scratch_types=[
            pltpu.VMEM((n_chunks, CHUNK), jnp.int32),
            pltpu.VMEM((CHUNK, DH), jnp.float32),
            pltpu.VMEM_SHARED((NDST, DH), jnp.float32),
            pltpu.SemaphoreType.DMA,
            pltpu.SemaphoreType.DMA,
        ],
    )
    def count(dst_hbm, zeros_hbm, ones_hbm, out_hbm, didx, ones_v, acc,
              ssem0, ssem1):
        c = lax.axis_index("c")
        s = lax.axis_index("s")
        wid = c * NS + s
        pltpu.sync_copy(dst_hbm.at[wid], didx)
        pltpu.sync_copy(ones_hbm, ones_v)
        _strided_row_copy(s, n_rblk, lambda r0: pltpu.sync_copy(
            zeros_hbm.at[pl.ds(r0, RB)], acc.at[pl.ds(r0, RB)]))
        plsc.subcore_barrier()

        def scatter(j, sem):
            return pltpu.async_copy(ones_v, acc.at[didx.at[j]], sem, add=True)

        scatter(0, ssem0).wait()

        def body(g, carry):
            s0 = scatter(1 + 2 * g, ssem0)
            s1 = scatter(2 + 2 * g, ssem1)
            s0.wait()
            s1.wait()
            return carry

        lax.fori_loop(0, n_pairs, body, 0)
        plsc.subcore_barrier()
        _strided_row_copy(s, n_rblk, lambda r0: pltpu.sync_copy(
            acc.at[pl.ds(r0, RB)], out_hbm.at[c, pl.ds(r0, RB)]))

    return count


def _proj_body(x_ref, w_ref, b_ref, o_ref):
    y = jnp.dot(x_ref[...], w_ref[...], preferred_element_type=jnp.float32)
    o_ref[...] = y + b_ref[...]


def _project(x, w, b, block_rows):
    n = x.shape[0]
    return pl.pallas_call(
        _proj_body,
        grid=(n // block_rows,),
        in_specs=[
            pl.BlockSpec((block_rows, DIN), lambda i: (i, 0)),
            pl.BlockSpec((DIN, DH), lambda i: (0, 0)),
            pl.BlockSpec((1, DH), lambda i: (0, 0)),
        ],
        out_specs=pl.BlockSpec((block_rows, DH), lambda i: (i, 0)),
        out_shape=jax.ShapeDtypeStruct((n, DH), jnp.float32),
    )(x, w, b)


def _layer_body(T):
    def body(*refs):
        p_refs = refs[:T]
        c_refs = refs[T:2 * T]
        x_ref, wn_ref, wr_ref, b_ref, o_ref = refs[2 * T:]
        wr_sum = jnp.sum(wr_ref[...], axis=0)
        acc = jnp.dot(x_ref[...], wr_sum, preferred_element_type=jnp.float32)
        acc = acc + jnp.sum(b_ref[...], axis=0)[None, :]
        for t in range(T):
            p = p_refs[t][...]
            ssum = p[0] + p[1]
            cnt = c_refs[t][0, :, 0:1] + c_refs[t][1, :, 0:1]
            agg = ssum * (1.0 / jnp.maximum(cnt, 1.0))
            acc = acc + jnp.dot(agg, wn_ref[t], preferred_element_type=jnp.float32)
        o_ref[...] = jnp.maximum(acc, 0.0)

    return body


def _layer(partials, counts, x, wn, wr, b, block_rows):
    T = len(partials)
    n = x.shape[0]
    pc_specs = [
        pl.BlockSpec((NC, block_rows, DH), lambda i: (0, i, 0))
        for _ in range(2 * T)
    ]
    return pl.pallas_call(
        _layer_body(T),
        grid=(n // block_rows,),
        in_specs=pc_specs
        + [
            pl.BlockSpec((block_rows, DH), lambda i: (i, 0)),
            pl.BlockSpec((T, DH, DH), lambda i: (0, 0, 0)),
            pl.BlockSpec((T, DH, DH), lambda i: (0, 0, 0)),
            pl.BlockSpec((T, DH), lambda i: (0, 0)),
        ],
        out_specs=pl.BlockSpec((block_rows, DH), lambda i: (i, 0)),
        out_shape=jax.ShapeDtypeStruct((n, DH), jnp.float32),
    )(*partials, *counts, x, wn, wr, b)


def kernel(x_gene, x_drug, edge_index_ppi, edge_index_gsea, edge_index_pcc,
           edge_index_dds, edge_index_dti_dg, edge_index_dti_gd,
           Wg, bg, Wd, bd, Wl, bl, Wr):
    zeros_g = jnp.zeros((NG, DH), jnp.float32)
    zeros_d = jnp.zeros((ND, DH), jnp.float32)
    ones_c = jnp.ones((CHUNK, DH), jnp.float32)

    xg = _project(x_gene, Wg, bg.reshape(1, DH), 2000)
    xd = _project(x_drug, Wd, bd.reshape(1, DH), 2000)

    seg_gg = _sc_segment_sum(320000, NG, NG)
    seg_dg = _sc_segment_sum(64000, ND, NG)
    seg_dd = _sc_segment_sum(64000, ND, ND)
    seg_gd = _sc_segment_sum(64000, NG, ND)
    cnt_big = _sc_count(320000, NG)
    cnt_g = _sc_count(64000, NG)
    cnt_d = _sc_count(64000, ND)

    def split3(e):
        nw = NC * NS
        return e[0], e[1].reshape(nw, e.shape[1] // (nw * CHUNK), CHUNK)

    s_ppi, d_ppi = split3(edge_index_ppi)
    s_gsea, d_gsea = split3(edge_index_gsea)
    s_pcc, d_pcc = split3(edge_index_pcc)
    s_dds, d_dds = split3(edge_index_dds)
    s_dti_dg, d_dti_dg = split3(edge_index_dti_dg)
    s_dti_gd, d_dti_gd = split3(edge_index_dti_gd)

    c_ppi = cnt_big(d_ppi, zeros_g, ones_c)
    c_gsea = cnt_big(d_gsea, zeros_g, ones_c)
    c_pcc = cnt_big(d_pcc, zeros_g, ones_c)
    c_dti_dg = cnt_g(d_dti_dg, zeros_g, ones_c)
    c_dds = cnt_d(d_dds, zeros_d, ones_c)
    c_dti_gd = cnt_d(d_dti_gd, zeros_d, ones_c)

    for l in range(L):
        p_ppi = seg_gg(xg, s_ppi, d_ppi, zeros_g)
        p_gsea = seg_gg(xg, s_gsea, d_gsea, zeros_g)
        p_pcc = seg_gg(xg, s_pcc, d_pcc, zeros_g)
        p_dti_dg = seg_dg(xd, s_dti_dg, d_dti_dg, zeros_g)
        p_dds = seg_dd(xd, s_dds, d_dds, zeros_d)
        p_dti_gd = seg_gd(xg, s_dti_gd, d_dti_gd, zeros_d)

        wn_g = jnp.stack([Wl[l, 0], Wl[l, 1], Wl[l, 2], Wl[l, 4]])
        wr_g = jnp.stack([Wr[l, 0], Wr[l, 1], Wr[l, 2], Wr[l, 4]])
        b_g = jnp.stack([bl[l, 0], bl[l, 1], bl[l, 2], bl[l, 4]])
        wn_d = jnp.stack([Wl[l, 3], Wl[l, 5]])
        wr_d = jnp.stack([Wr[l, 3], Wr[l, 5]])
        b_d = jnp.stack([bl[l, 3], bl[l, 5]])

        xg = _layer([p_ppi, p_gsea, p_pcc, p_dti_dg],
                    [c_ppi, c_gsea, c_pcc, c_dti_dg],
                    xg, wn_g, wr_g, b_g, 2000)
        xd = _layer([p_dds, p_dti_gd], [c_dds, c_dti_gd],
                    xd, wn_d, wr_d, b_d, 2000)

    return xg, xd
